# baseline (speedup 1.0000x reference)
import functools

import jax
import jax.numpy as jnp
from jax import lax
from jax.experimental import pallas as pl
from jax.experimental.pallas import tpu as pltpu
from jax.experimental.pallas import tpu_sc as plsc

_F32 = jnp.float32
_BF16 = jnp.bfloat16
_I32 = jnp.int32
_COST_LAMBDA = 0.0005

_NC, _NS, _LANES = 2, 16, 16
_NW = _NC * _NS


def _rup128(n):
    return (n + 127) // 128 * 128


def _gelu(v):
    return 0.5 * v * (1.0 + jax.lax.erf(v * (2.0 ** -0.5)))


def _dot_t(a, b):
    return jax.lax.dot_general(a, b, (((1,), (1,)), ((), ())),
                               preferred_element_type=_F32)


def _dot(a, b):
    return jax.lax.dot_general(a, b, (((1,), (0,)), ((), ())),
                               preferred_element_type=_F32)


def _k1_body(x_ref, rmsw_ref, wqkv_ref, wd_ref, bd_ref, wr_ref, breff_ref,
             h_ref, q_ref, k_ref, v_ref, g_ref, w1c_ref, w2c_ref):
    xb = x_ref[...]
    d = xb.shape[-1]
    norm = jnp.sqrt(jnp.sum(xb * xb, axis=-1, keepdims=True)) * (d ** -0.5)
    hb = rmsw_ref[...] * xb / (norm + 1e-8)
    h_ref[...] = hb

    qkv = _dot_t(hb, wqkv_ref[...])
    q_ref[...] = qkv[:, :d].astype(_BF16)
    k_ref[...] = qkv[:, d:2 * d].astype(_BF16)
    v_ref[...] = qkv[:, 2 * d:].astype(_BF16)

    hd = _dot_t(hb, wd_ref[...]) + bd_ref[...]
    g_ref[...] = _gelu(hd).astype(_BF16)

    logits = _dot_t(hb, wr_ref[...]) + breff_ref[...]
    e = logits.shape[-1]
    iota = jax.lax.broadcasted_iota(jnp.int32, logits.shape, 1)
    l1 = jnp.max(logits, axis=-1, keepdims=True)
    a1 = jnp.min(jnp.where(logits == l1, iota, e), axis=-1, keepdims=True)
    masked = jnp.where(iota == a1, -jnp.inf, logits)
    l2 = jnp.max(masked, axis=-1, keepdims=True)
    a2 = jnp.min(jnp.where(masked == l2, iota, e), axis=-1, keepdims=True)
    z = jnp.sum(jnp.exp(logits - l1), axis=-1, keepdims=True)
    p1 = 1.0 / z
    p2 = jnp.exp(l2 - l1) / z
    e2 = jnp.exp(p2 - p1)
    inv = 1.0 / (1.0 + e2)
    w1c_ref[...] = jnp.where(iota == a1, inv, 0.0)
    w2c_ref[...] = jnp.where(iota == a2, e2 * inv, 0.0)


def _k1b_body(w1c_ref, w2c_ref, b1a_ref, w2a_ref, b2a_ref,
              dst_ref, wt_ref, pos_ref, be_ref, ctot_ref,
              *, n_tok, n_exp, re_blk, n_blk):
    eye = (jax.lax.broadcasted_iota(_I32, (n_exp, n_exp), 0) ==
           jax.lax.broadcasted_iota(_I32, (n_exp, n_exp), 1)).astype(_F32)
    w1t = _dot_t(eye, w1c_ref[...])
    w2t = _dot_t(eye, w2c_ref[...])
    wt = w1t + w2t
    wt_ref[...] = wt

    m = (wt > 0.0).astype(_F32)
    incl = m
    s = 1
    while s < n_tok:
        shifted = jnp.concatenate(
            [jnp.zeros((n_exp, s), _F32), incl[:, :n_tok - s]], axis=1)
        incl = incl + shifted
        s *= 2
    cnt = incl[:, n_tok - 1:n_tok]
    pcnt = jnp.floor((cnt + (re_blk - 1)) * (1.0 / re_blk)) * re_blk
    mlt = (jax.lax.broadcasted_iota(_I32, (n_exp, n_exp), 1) <
           jax.lax.broadcasted_iota(_I32, (n_exp, n_exp), 0)).astype(_F32)
    offp = _dot(mlt, pcnt)
    dstf = offp + incl - 1.0
    dst_ref[...] = dstf.astype(_I32)

    pos_ref[0:1, :] = jnp.sum((w1t > 0.0).astype(_F32) * dstf, axis=0,
                              keepdims=True).astype(_I32)
    pos_ref[1:2, :] = jnp.sum((w2t > 0.0).astype(_F32) * dstf, axis=0,
                              keepdims=True).astype(_I32)

    biota = jax.lax.broadcasted_iota(_I32, (n_exp, n_blk), 1).astype(_F32) * re_blk
    be_ref[...] = (jnp.sum((biota >= offp).astype(_F32), axis=0,
                           keepdims=True) - 1.0).astype(_I32)

    ctot = jnp.zeros((1, ctot_ref.shape[1]), _F32)
    for i in range(n_exp):
        gb1 = _gelu(b1a_ref[i])
        ctot = ctot + _dot_t(gb1, w2a_ref[i]) + b2a_ref[i]
    ctot_ref[...] = ctot


def _sca_body(dst_hbm, gate_hbm, tok_out, gate_out,
              dstv, gatev, tokl, gatel, *, npad, n_entries, tokmask):
    own = npad // _NW
    wid = lax.axis_index("s") * _NC + lax.axis_index("c")
    lo = wid * own

    zi = jnp.zeros((_LANES,), _I32)
    zf = jnp.zeros((_LANES,), _F32)

    @pl.loop(0, own // _LANES)
    def _init(i):
        tokl[pl.ds(i * _LANES, _LANES)] = zi
        gatel[pl.ds(i * _LANES, _LANES)] = zf

    lane_iota = jax.lax.iota(_I32, _LANES)
    ch = dstv.shape[0]

    @pl.loop(0, n_entries // ch)
    def _chunk(c):
        base = c * ch
        pltpu.sync_copy(dst_hbm.at[pl.ds(base, ch)], dstv)
        pltpu.sync_copy(gate_hbm.at[pl.ds(base, ch)], gatev)

        @pl.loop(0, ch // _LANES)
        def _scan(i):
            dv = dstv[pl.ds(i * _LANES, _LANES)]
            gv = gatev[pl.ds(i * _LANES, _LANES)]
            lidx = dv - lo
            ok = (gv > 0.0) & (lidx >= 0) & (lidx < own)
            lidx = jnp.where(ok, lidx, 0)
            tid = jnp.bitwise_and(base + i * _LANES + lane_iota, tokmask)
            plsc.store_scatter(tokl, [lidx], tid, mask=ok)
            plsc.store_scatter(gatel, [lidx], gv, mask=ok)

    pltpu.sync_copy(tokl.at[pl.ds(0, own)], tok_out.at[pl.ds(lo, own)])
    pltpu.sync_copy(gatel.at[pl.ds(0, own)], gate_out.at[pl.ds(lo, own)])


def _scg_body(table_hbm, idx_hbm, out_hbm, idxv, rows, sem, *, n_rows, nmax):
    own = n_rows // _NW
    ch = idxv.shape[0]
    wid = lax.axis_index("s") * _NC + lax.axis_index("c")
    base = wid * own
    for c in range(own // ch):
        pltpu.sync_copy(idx_hbm.at[pl.ds(base + c * ch, ch)], idxv)
        if nmax is not None:
            @pl.loop(0, ch // _LANES)
            def _san(i):
                v = idxv[pl.ds(i * _LANES, _LANES)]
                idxv[pl.ds(i * _LANES, _LANES)] = jnp.minimum(
                    jnp.maximum(v, 0), nmax)
        pltpu.async_copy(table_hbm.at[idxv], rows, sem).wait()
        pltpu.sync_copy(rows, out_hbm.at[pl.ds(base + c * ch, ch)])


def _tce_body(be_ref, gbuf_ref, gate_ref, w1_ref, b1_ref, w2_ref, out_ref):
    b1 = b1_ref[0]
    z = _dot_t(gbuf_ref[...], w1_ref[0]) + b1
    t = _gelu(z) - _gelu(b1)
    out_ref[...] = _dot_t(gate_ref[...] * t, w2_ref[0]).astype(_BF16)


def _k2_body(q_ref, k_ref, v_ref, x_ref, wo_ref, o_ref, *, scale):
    qb = q_ref[0]
    scores = _dot_t(qb, k_ref[0]) * scale
    m = jnp.max(scores, axis=-1, keepdims=True)
    p = jnp.exp(scores - m)
    att = p / jnp.sum(p, axis=-1, keepdims=True)
    o = _dot(att, v_ref[0])
    o_ref[0] = x_ref[0] + _dot_t(o, wo_ref[...])


def _k3_body(y1_ref, h_ref, g_ref, y2a_ref, y2b_ref, ctot_ref,
             ws1_ref, bs1_ref, ws2_ref, bs2_ref, wu_ref, bu_ref,
             wc_ref, bc_ref, out_ref):
    gb = g_ref[...]
    s = _gelu(_dot_t(gb, ws1_ref[...]) + bs1_ref[...])
    s = _dot_t(s, ws2_ref[...]) + bs2_ref[...]
    moe = y2a_ref[...] + y2b_ref[...] + ctot_ref[...] + 0.1 * s

    up = _dot_t(moe, wu_ref[...]) + bu_ref[...]
    core = _dot_t(_gelu(h_ref[...]), wc_ref[...]) + bc_ref[...]
    out_ref[...] = y1_ref[...] + up + core


def kernel(x, rms_w, Wqkv, Wo, Wd, bd, Wu, bu, Wr, br, expert_params,
           Ws1, bs1, Ws2, bs2, Wc, bc):
    B, T, D = x.shape
    N = B * T
    L = Wd.shape[0]
    E = Wr.shape[0]
    hdims = [int(w1.shape[0]) for (w1, _, _, _) in expert_params]
    hpad = max(hdims)
    cost = jnp.asarray([2 * L * hd for hd in hdims], _F32)
    br_eff = (br - _COST_LAMBDA * cost).reshape(1, E)

    RE = 256
    NPAD = 2 * N + E * RE
    NBLK = NPAD // RE

    W1all = jnp.stack([jnp.pad(w1, ((0, hpad - hd), (0, 0)))
                       for (w1, _, _, _), hd in zip(expert_params, hdims)])
    b1all = jnp.stack([jnp.pad(b1, (0, hpad - hd)).reshape(1, hpad)
                       for (_, b1, _, _), hd in zip(expert_params, hdims)])
    W2all = jnp.stack([jnp.pad(w2, ((0, 0), (0, hpad - hd)))
                       for (_, _, w2, _), hd in zip(expert_params, hdims)])
    b2all = jnp.stack([b2.reshape(1, L) for (_, _, _, b2) in expert_params])

    x2 = x.reshape(N, D)
    r2 = lambda a: a.reshape(1, -1)

    R1 = 512
    full = lambda arr: pl.BlockSpec(arr.shape, lambda i: (0,) * arr.ndim)
    row = lambda c: pl.BlockSpec((R1, c), lambda i: (i, 0))

    h, q, k, v, g, w1c, w2c = pl.pallas_call(
        _k1_body,
        grid=(N // R1,),
        in_specs=[row(D), full(r2(rms_w)), full(Wqkv), full(Wd),
                  full(r2(bd)), full(Wr), full(br_eff)],
        out_specs=[row(D), row(D), row(D), row(D), row(L), row(E), row(E)],
        out_shape=[
            jax.ShapeDtypeStruct((N, D), _F32),
            jax.ShapeDtypeStruct((N, D), _BF16),
            jax.ShapeDtypeStruct((N, D), _BF16),
            jax.ShapeDtypeStruct((N, D), _BF16),
            jax.ShapeDtypeStruct((N, L), _BF16),
            jax.ShapeDtypeStruct((N, E), _F32),
            jax.ShapeDtypeStruct((N, E), _F32),
        ],
    )(x2, r2(rms_w), Wqkv, Wd, r2(bd), Wr, br_eff)

    dst8, wt8, pos2, be, ctot = pl.pallas_call(
        functools.partial(_k1b_body, n_tok=N, n_exp=E, re_blk=RE, n_blk=NBLK),
        grid=(1,),
        in_specs=[full(w1c), full(w2c), full(b1all), full(W2all), full(b2all)],
        out_specs=[
            pl.BlockSpec((E, N), lambda i: (0, 0)),
            pl.BlockSpec((E, N), lambda i: (0, 0)),
            pl.BlockSpec((2, N), lambda i: (0, 0)),
            pl.BlockSpec((1, NBLK), lambda i: (0, 0)),
            pl.BlockSpec((1, L), lambda i: (0, 0)),
        ],
        out_shape=[
            jax.ShapeDtypeStruct((E, N), _I32),
            jax.ShapeDtypeStruct((E, N), _F32),
            jax.ShapeDtypeStruct((2, N), _I32),
            jax.ShapeDtypeStruct((1, NBLK), _I32),
            jax.ShapeDtypeStruct((1, L), _F32),
        ],
    )(w1c, w2c, b1all, W2all, b2all)

    mesh = plsc.VectorSubcoreMesh(core_axis_name="c", subcore_axis_name="s")

    NENT = E * N
    SCCH = 4096
    sca = pl.kernel(
        functools.partial(_sca_body, npad=NPAD, n_entries=NENT,
                          tokmask=N - 1),
        out_type=[jax.ShapeDtypeStruct((NPAD,), _I32),
                  jax.ShapeDtypeStruct((NPAD,), _F32)],
        mesh=mesh,
        scratch_types=[pltpu.VMEM((SCCH,), _I32),
                       pltpu.VMEM((SCCH,), _F32),
                       pltpu.VMEM((_rup128(NPAD // _NW),), _I32),
                       pltpu.VMEM((_rup128(NPAD // _NW),), _F32)],
        compiler_params=pltpu.CompilerParams(needs_layout_passes=False),
    )
    tok, gateb = sca(dst8.reshape(NENT), wt8.reshape(NENT))

    RQ = 512
    q3 = q.reshape(B, T, D)
    k3 = k.reshape(B, T, D)
    v3 = v.reshape(B, T, D)
    qblk = pl.BlockSpec((1, RQ, D), lambda b, i: (b, i, 0))
    kvblk = pl.BlockSpec((1, T, D), lambda b, i: (b, 0, 0))
    y1 = pl.pallas_call(
        functools.partial(_k2_body, scale=D ** -0.5),
        grid=(B, T // RQ),
        in_specs=[qblk, kvblk, kvblk, qblk,
                  pl.BlockSpec(Wo.shape, lambda b, i: (0, 0))],
        out_specs=qblk,
        out_shape=jax.ShapeDtypeStruct((B, T, D), _F32),
    )(q3, k3, v3, x, Wo)

    LP = L // 2
    gp = jax.lax.bitcast_convert_type(g.reshape(N, LP, 2), _F32)
    scb = pl.kernel(
        functools.partial(_scg_body, n_rows=NPAD, nmax=None),
        out_type=jax.ShapeDtypeStruct((NPAD, LP), _F32),
        mesh=mesh,
        scratch_types=[pltpu.VMEM((NPAD // _NW // 2,), _I32),
                       pltpu.VMEM((NPAD // _NW // 2, LP), _F32),
                       pltpu.SemaphoreType.DMA],
    )
    gbuf = jax.lax.bitcast_convert_type(
        scb(gp, tok), _BF16).reshape(NPAD, L)

    ybuf = pl.pallas_call(
        _tce_body,
        grid_spec=pltpu.PrefetchScalarGridSpec(
            num_scalar_prefetch=1,
            grid=(NBLK,),
            in_specs=[
                pl.BlockSpec((RE, L), lambda b, be_s: (b, 0)),
                pl.BlockSpec((RE, 1), lambda b, be_s: (b, 0)),
                pl.BlockSpec((1, hpad, L), lambda b, be_s: (be_s[b], 0, 0)),
                pl.BlockSpec((1, 1, hpad), lambda b, be_s: (be_s[b], 0, 0)),
                pl.BlockSpec((1, L, hpad), lambda b, be_s: (be_s[b], 0, 0)),
            ],
            out_specs=pl.BlockSpec((RE, L), lambda b, be_s: (b, 0)),
        ),
        out_shape=jax.ShapeDtypeStruct((NPAD, L), _BF16),
    )(be.reshape(NBLK), gbuf, gateb.reshape(NPAD, 1), W1all, b1all, W2all)

    ybp = jax.lax.bitcast_convert_type(ybuf.reshape(NPAD, LP, 2), _F32)
    scc = pl.kernel(
        functools.partial(_scg_body, n_rows=2 * N, nmax=None),
        out_type=jax.ShapeDtypeStruct((2 * N, LP), _F32),
        mesh=mesh,
        scratch_types=[pltpu.VMEM((2 * N // _NW // 2,), _I32),
                       pltpu.VMEM((2 * N // _NW // 2, LP), _F32),
                       pltpu.SemaphoreType.DMA],
    )
    y2 = jax.lax.bitcast_convert_type(
        scc(ybp, pos2.reshape(2 * N)), _BF16).reshape(2 * N, L)

    R3 = 512
    row3 = lambda c: pl.BlockSpec((R3, c), lambda i: (i, 0))
    out = pl.pallas_call(
        _k3_body,
        grid=(N // R3,),
        in_specs=[row3(D), row3(D), row3(L), row3(L), row3(L),
                  full(ctot),
                  full(Ws1), full(r2(bs1)), full(Ws2), full(r2(bs2)),
                  full(Wu), full(r2(bu)), full(Wc), full(r2(bc))],
        out_specs=row3(D),
        out_shape=jax.ShapeDtypeStruct((N, D), _F32),
    )(y1.reshape(N, D), h, g, y2[:N], y2[N:], ctot,
      Ws1, r2(bs1), Ws2, r2(bs2), Wu, r2(bu), Wc, r2(bc))

    return out.reshape(B, T, D)


# speedup vs baseline: 1.6902x; 1.6902x over previous
import functools

import jax
import jax.numpy as jnp
from jax import lax
from jax.experimental import pallas as pl
from jax.experimental.pallas import tpu as pltpu
from jax.experimental.pallas import tpu_sc as plsc

_F32 = jnp.float32
_BF16 = jnp.bfloat16
_I32 = jnp.int32
_COST_LAMBDA = 0.0005

_NC, _NS, _LANES = 2, 16, 16
_NW = _NC * _NS


def _rup128(n):
    return (n + 127) // 128 * 128


def _gelu(v):
    return 0.5 * v * (1.0 + jax.lax.erf(v * (2.0 ** -0.5)))


def _dot_t(a, b):
    return jax.lax.dot_general(a, b, (((1,), (1,)), ((), ())),
                               preferred_element_type=_F32)


def _dot(a, b):
    return jax.lax.dot_general(a, b, (((1,), (0,)), ((), ())),
                               preferred_element_type=_F32)


def _k1_body(x_ref, rmsw_ref, wqkv_ref, wd_ref, bd_ref, wr_ref, breff_ref,
             h_ref, q_ref, k_ref, v_ref, g_ref, w1c_ref, w2c_ref):
    xb = x_ref[...]
    d = xb.shape[-1]
    norm = jnp.sqrt(jnp.sum(xb * xb, axis=-1, keepdims=True)) * (d ** -0.5)
    hb = rmsw_ref[...] * xb / (norm + 1e-8)
    h_ref[...] = hb

    qkv = _dot_t(hb, wqkv_ref[...])
    q_ref[...] = qkv[:, :d].astype(_BF16)
    k_ref[...] = qkv[:, d:2 * d].astype(_BF16)
    v_ref[...] = qkv[:, 2 * d:].astype(_BF16)

    hd = _dot_t(hb, wd_ref[...]) + bd_ref[...]
    g_ref[...] = _gelu(hd)

    logits = _dot_t(hb, wr_ref[...]) + breff_ref[...]
    e = logits.shape[-1]
    iota = jax.lax.broadcasted_iota(jnp.int32, logits.shape, 1)
    l1 = jnp.max(logits, axis=-1, keepdims=True)
    a1 = jnp.min(jnp.where(logits == l1, iota, e), axis=-1, keepdims=True)
    masked = jnp.where(iota == a1, -jnp.inf, logits)
    l2 = jnp.max(masked, axis=-1, keepdims=True)
    a2 = jnp.min(jnp.where(masked == l2, iota, e), axis=-1, keepdims=True)
    z = jnp.sum(jnp.exp(logits - l1), axis=-1, keepdims=True)
    p1 = 1.0 / z
    p2 = jnp.exp(l2 - l1) / z
    e2 = jnp.exp(p2 - p1)
    inv = 1.0 / (1.0 + e2)
    w1c_ref[...] = jnp.where(iota == a1, inv, 0.0)
    w2c_ref[...] = jnp.where(iota == a2, e2 * inv, 0.0)


def _k1b_body(w1c_ref, w2c_ref, b1a_ref, w2a_ref, b2a_ref,
              dst_ref, wt_ref, pos_ref, be_ref, ctot_ref,
              *, n_tok, n_exp, re_blk, n_blk):
    eye = (jax.lax.broadcasted_iota(_I32, (n_exp, n_exp), 0) ==
           jax.lax.broadcasted_iota(_I32, (n_exp, n_exp), 1)).astype(_F32)
    w1t = _dot_t(eye, w1c_ref[...])
    w2t = _dot_t(eye, w2c_ref[...])
    wt = w1t + w2t
    wt_ref[...] = wt

    m = (wt > 0.0).astype(_F32)
    incl = m
    s = 1
    while s < n_tok:
        shifted = jnp.concatenate(
            [jnp.zeros((n_exp, s), _F32), incl[:, :n_tok - s]], axis=1)
        incl = incl + shifted
        s *= 2
    cnt = incl[:, n_tok - 1:n_tok]
    pcnt = jnp.floor((cnt + (re_blk - 1)) * (1.0 / re_blk)) * re_blk
    mlt = (jax.lax.broadcasted_iota(_I32, (n_exp, n_exp), 1) <
           jax.lax.broadcasted_iota(_I32, (n_exp, n_exp), 0)).astype(_F32)
    offp = _dot(mlt, pcnt)
    dstf = offp + incl - 1.0
    dst_ref[...] = dstf.astype(_I32)

    pos_ref[0:1, :] = jnp.sum((w1t > 0.0).astype(_F32) * dstf, axis=0,
                              keepdims=True).astype(_I32)
    pos_ref[1:2, :] = jnp.sum((w2t > 0.0).astype(_F32) * dstf, axis=0,
                              keepdims=True).astype(_I32)

    biota = jax.lax.broadcasted_iota(_I32, (n_exp, n_blk), 1).astype(_F32) * re_blk
    be_ref[...] = (jnp.sum((biota >= offp).astype(_F32), axis=0,
                           keepdims=True) - 1.0).astype(_I32)

    ctot = jnp.zeros((1, ctot_ref.shape[1]), _F32)
    for i in range(n_exp):
        gb1 = _gelu(b1a_ref[i])
        ctot = ctot + _dot_t(gb1, w2a_ref[i]) + b2a_ref[i]
    ctot_ref[...] = ctot


def _sca_body(dst_hbm, gate_hbm, tok_out, gate_out,
              dstv, gatev, tokl, gatel, *, npad, n_entries, tokmask):
    own = npad // _NW
    wid = lax.axis_index("s") * _NC + lax.axis_index("c")
    lo = wid * own

    zi = jnp.zeros((_LANES,), _I32)
    zf = jnp.zeros((_LANES,), _F32)

    @pl.loop(0, own // _LANES)
    def _init(i):
        tokl[pl.ds(i * _LANES, _LANES)] = zi
        gatel[pl.ds(i * _LANES, _LANES)] = zf

    lane_iota = jax.lax.iota(_I32, _LANES)
    ch = dstv.shape[0]

    @pl.loop(0, n_entries // ch)
    def _chunk(c):
        base = c * ch
        pltpu.sync_copy(dst_hbm.at[pl.ds(base, ch)], dstv)
        pltpu.sync_copy(gate_hbm.at[pl.ds(base, ch)], gatev)

        @pl.loop(0, ch // _LANES)
        def _scan(i):
            dv = dstv[pl.ds(i * _LANES, _LANES)]
            gv = gatev[pl.ds(i * _LANES, _LANES)]
            lidx = dv - lo
            ok = (gv > 0.0) & (lidx >= 0) & (lidx < own)
            lidx = jnp.where(ok, lidx, 0)
            tid = jnp.bitwise_and(base + i * _LANES + lane_iota, tokmask)
            plsc.store_scatter(tokl, [lidx], tid, mask=ok)
            plsc.store_scatter(gatel, [lidx], gv, mask=ok)

    pltpu.sync_copy(tokl.at[pl.ds(0, own)], tok_out.at[pl.ds(lo, own)])
    pltpu.sync_copy(gatel.at[pl.ds(0, own)], gate_out.at[pl.ds(lo, own)])


def _scg_body(table_hbm, idx_hbm, out_hbm, idxv, rows, sem, *, n_rows, nmax):
    own = n_rows // _NW
    ch = idxv.shape[0]
    wid = lax.axis_index("s") * _NC + lax.axis_index("c")
    base = wid * own
    for c in range(own // ch):
        pltpu.sync_copy(idx_hbm.at[pl.ds(base + c * ch, ch)], idxv)
        if nmax is not None:
            @pl.loop(0, ch // _LANES)
            def _san(i):
                v = idxv[pl.ds(i * _LANES, _LANES)]
                idxv[pl.ds(i * _LANES, _LANES)] = jnp.minimum(
                    jnp.maximum(v, 0), nmax)
        pltpu.async_copy(table_hbm.at[idxv], rows, sem).wait()
        pltpu.sync_copy(rows, out_hbm.at[pl.ds(base + c * ch, ch)])


def _tce_body(be_ref, gbuf_ref, gate_ref, w1_ref, b1_ref, w2_ref, out_ref):
    b1 = b1_ref[0]
    z = _dot_t(gbuf_ref[...], w1_ref[0]) + b1
    t = _gelu(z) - _gelu(b1)
    out_ref[...] = _dot_t(gate_ref[...] * t, w2_ref[0])


def _k2_body(q_ref, k_ref, v_ref, x_ref, wo_ref, o_ref, *, scale):
    qb = q_ref[0]
    scores = _dot_t(qb, k_ref[0]) * scale
    m = jnp.max(scores, axis=-1, keepdims=True)
    p = jnp.exp(scores - m)
    att = p / jnp.sum(p, axis=-1, keepdims=True)
    o = _dot(att, v_ref[0])
    o_ref[0] = x_ref[0] + _dot_t(o, wo_ref[...])


def _k3_body(y1_ref, h_ref, g_ref, y2a_ref, y2b_ref, ctot_ref,
             ws1_ref, bs1_ref, ws2_ref, bs2_ref, wu_ref, bu_ref,
             wc_ref, bc_ref, out_ref):
    gb = g_ref[...]
    s = _gelu(_dot_t(gb, ws1_ref[...]) + bs1_ref[...])
    s = _dot_t(s, ws2_ref[...]) + bs2_ref[...]
    moe = y2a_ref[...] + y2b_ref[...] + ctot_ref[...] + 0.1 * s

    up = _dot_t(moe, wu_ref[...]) + bu_ref[...]
    core = _dot_t(_gelu(h_ref[...]), wc_ref[...]) + bc_ref[...]
    out_ref[...] = y1_ref[...] + up + core


def kernel(x, rms_w, Wqkv, Wo, Wd, bd, Wu, bu, Wr, br, expert_params,
           Ws1, bs1, Ws2, bs2, Wc, bc):
    B, T, D = x.shape
    N = B * T
    L = Wd.shape[0]
    E = Wr.shape[0]
    hdims = [int(w1.shape[0]) for (w1, _, _, _) in expert_params]
    hpad = max(hdims)
    cost = jnp.asarray([2 * L * hd for hd in hdims], _F32)
    br_eff = (br - _COST_LAMBDA * cost).reshape(1, E)

    RE = 256
    NPAD = 2 * N + E * RE
    NBLK = NPAD // RE

    W1all = jnp.stack([jnp.pad(w1, ((0, hpad - hd), (0, 0)))
                       for (w1, _, _, _), hd in zip(expert_params, hdims)])
    b1all = jnp.stack([jnp.pad(b1, (0, hpad - hd)).reshape(1, hpad)
                       for (_, b1, _, _), hd in zip(expert_params, hdims)])
    W2all = jnp.stack([jnp.pad(w2, ((0, 0), (0, hpad - hd)))
                       for (_, _, w2, _), hd in zip(expert_params, hdims)])
    b2all = jnp.stack([b2.reshape(1, L) for (_, _, _, b2) in expert_params])

    x2 = x.reshape(N, D)
    r2 = lambda a: a.reshape(1, -1)

    R1 = 512
    full = lambda arr: pl.BlockSpec(arr.shape, lambda i: (0,) * arr.ndim)
    row = lambda c: pl.BlockSpec((R1, c), lambda i: (i, 0))

    h, q, k, v, g, w1c, w2c = pl.pallas_call(
        _k1_body,
        grid=(N // R1,),
        in_specs=[row(D), full(r2(rms_w)), full(Wqkv), full(Wd),
                  full(r2(bd)), full(Wr), full(br_eff)],
        out_specs=[row(D), row(D), row(D), row(D), row(L), row(E), row(E)],
        out_shape=[
            jax.ShapeDtypeStruct((N, D), _F32),
            jax.ShapeDtypeStruct((N, D), _BF16),
            jax.ShapeDtypeStruct((N, D), _BF16),
            jax.ShapeDtypeStruct((N, D), _BF16),
            jax.ShapeDtypeStruct((N, L), _F32),
            jax.ShapeDtypeStruct((N, E), _F32),
            jax.ShapeDtypeStruct((N, E), _F32),
        ],
    )(x2, r2(rms_w), Wqkv, Wd, r2(bd), Wr, br_eff)

    dst8, wt8, pos2, be, ctot = pl.pallas_call(
        functools.partial(_k1b_body, n_tok=N, n_exp=E, re_blk=RE, n_blk=NBLK),
        grid=(1,),
        in_specs=[full(w1c), full(w2c), full(b1all), full(W2all), full(b2all)],
        out_specs=[
            pl.BlockSpec((E, N), lambda i: (0, 0)),
            pl.BlockSpec((E, N), lambda i: (0, 0)),
            pl.BlockSpec((2, N), lambda i: (0, 0)),
            pl.BlockSpec((1, NBLK), lambda i: (0, 0)),
            pl.BlockSpec((1, L), lambda i: (0, 0)),
        ],
        out_shape=[
            jax.ShapeDtypeStruct((E, N), _I32),
            jax.ShapeDtypeStruct((E, N), _F32),
            jax.ShapeDtypeStruct((2, N), _I32),
            jax.ShapeDtypeStruct((1, NBLK), _I32),
            jax.ShapeDtypeStruct((1, L), _F32),
        ],
    )(w1c, w2c, b1all, W2all, b2all)

    mesh = plsc.VectorSubcoreMesh(core_axis_name="c", subcore_axis_name="s")

    NENT = E * N
    SCCH = 4096
    sca = pl.kernel(
        functools.partial(_sca_body, npad=NPAD, n_entries=NENT,
                          tokmask=N - 1),
        out_type=[jax.ShapeDtypeStruct((NPAD,), _I32),
                  jax.ShapeDtypeStruct((NPAD,), _F32)],
        mesh=mesh,
        scratch_types=[pltpu.VMEM((SCCH,), _I32),
                       pltpu.VMEM((SCCH,), _F32),
                       pltpu.VMEM((_rup128(NPAD // _NW),), _I32),
                       pltpu.VMEM((_rup128(NPAD // _NW),), _F32)],
        compiler_params=pltpu.CompilerParams(needs_layout_passes=False),
    )
    tok, gateb = sca(dst8.reshape(NENT), wt8.reshape(NENT))

    RQ = 512
    q3 = q.reshape(B, T, D)
    k3 = k.reshape(B, T, D)
    v3 = v.reshape(B, T, D)
    qblk = pl.BlockSpec((1, RQ, D), lambda b, i: (b, i, 0))
    kvblk = pl.BlockSpec((1, T, D), lambda b, i: (b, 0, 0))
    y1 = pl.pallas_call(
        functools.partial(_k2_body, scale=D ** -0.5),
        grid=(B, T // RQ),
        in_specs=[qblk, kvblk, kvblk, qblk,
                  pl.BlockSpec(Wo.shape, lambda b, i: (0, 0))],
        out_specs=qblk,
        out_shape=jax.ShapeDtypeStruct((B, T, D), _F32),
    )(q3, k3, v3, x, Wo)

    scb = pl.kernel(
        functools.partial(_scg_body, n_rows=NPAD, nmax=None),
        out_type=jax.ShapeDtypeStruct((NPAD, L), _F32),
        mesh=mesh,
        scratch_types=[pltpu.VMEM((NPAD // _NW // 2,), _I32),
                       pltpu.VMEM((NPAD // _NW // 2, L), _F32),
                       pltpu.SemaphoreType.DMA],
    )
    gbuf = scb(g, tok)

    ybuf = pl.pallas_call(
        _tce_body,
        grid_spec=pltpu.PrefetchScalarGridSpec(
            num_scalar_prefetch=1,
            grid=(NBLK,),
            in_specs=[
                pl.BlockSpec((RE, L), lambda b, be_s: (b, 0)),
                pl.BlockSpec((RE, 1), lambda b, be_s: (b, 0)),
                pl.BlockSpec((1, hpad, L), lambda b, be_s: (be_s[b], 0, 0)),
                pl.BlockSpec((1, 1, hpad), lambda b, be_s: (be_s[b], 0, 0)),
                pl.BlockSpec((1, L, hpad), lambda b, be_s: (be_s[b], 0, 0)),
            ],
            out_specs=pl.BlockSpec((RE, L), lambda b, be_s: (b, 0)),
        ),
        out_shape=jax.ShapeDtypeStruct((NPAD, L), _F32),
    )(be.reshape(NBLK), gbuf, gateb.reshape(NPAD, 1), W1all, b1all, W2all)

    scc = pl.kernel(
        functools.partial(_scg_body, n_rows=2 * N, nmax=None),
        out_type=jax.ShapeDtypeStruct((2 * N, L), _F32),
        mesh=mesh,
        scratch_types=[pltpu.VMEM((2 * N // _NW // 2,), _I32),
                       pltpu.VMEM((2 * N // _NW // 2, L), _F32),
                       pltpu.SemaphoreType.DMA],
    )
    y2 = scc(ybuf, pos2.reshape(2 * N))

    R3 = 512
    row3 = lambda c: pl.BlockSpec((R3, c), lambda i: (i, 0))
    out = pl.pallas_call(
        _k3_body,
        grid=(N // R3,),
        in_specs=[row3(D), row3(D), row3(L), row3(L), row3(L),
                  full(ctot),
                  full(Ws1), full(r2(bs1)), full(Ws2), full(r2(bs2)),
                  full(Wu), full(r2(bu)), full(Wc), full(r2(bc))],
        out_specs=row3(D),
        out_shape=jax.ShapeDtypeStruct((N, D), _F32),
    )(y1.reshape(N, D), h, g, y2[:N], y2[N:], ctot,
      Ws1, r2(bs1), Ws2, r2(bs2), Wu, r2(bu), Wc, r2(bc))

    return out.reshape(B, T, D)


# speedup vs baseline: 1.7534x; 1.0374x over previous
import functools

import jax
import jax.numpy as jnp
from jax import lax
from jax.experimental import pallas as pl
from jax.experimental.pallas import tpu as pltpu
from jax.experimental.pallas import tpu_sc as plsc

_F32 = jnp.float32
_BF16 = jnp.bfloat16
_I32 = jnp.int32
_COST_LAMBDA = 0.0005

_NC, _NS, _LANES = 2, 16, 16
_NW = _NC * _NS


def _rup128(n):
    return (n + 127) // 128 * 128


def _gelu(v):
    return 0.5 * v * (1.0 + jax.lax.erf(v * (2.0 ** -0.5)))


def _dot_t(a, b):
    return jax.lax.dot_general(a, b, (((1,), (1,)), ((), ())),
                               preferred_element_type=_F32)


def _dot(a, b):
    return jax.lax.dot_general(a, b, (((1,), (0,)), ((), ())),
                               preferred_element_type=_F32)


def _k1_body(x_ref, rmsw_ref, wqkv_ref, wd_ref, bd_ref, wr_ref, breff_ref,
             h_ref, q_ref, k_ref, v_ref, g_ref, w1c_ref, w2c_ref):
    xb = x_ref[...]
    d = xb.shape[-1]
    norm = jnp.sqrt(jnp.sum(xb * xb, axis=-1, keepdims=True)) * (d ** -0.5)
    hb = rmsw_ref[...] * xb / (norm + 1e-8)
    h_ref[...] = hb

    qkv = _dot_t(hb, wqkv_ref[...])
    q_ref[...] = qkv[:, :d].astype(_BF16)
    k_ref[...] = qkv[:, d:2 * d].astype(_BF16)
    v_ref[...] = qkv[:, 2 * d:].astype(_BF16)

    hd = _dot_t(hb, wd_ref[...]) + bd_ref[...]
    g_ref[...] = _gelu(hd)

    logits = _dot_t(hb, wr_ref[...]) + breff_ref[...]
    e = logits.shape[-1]
    iota = jax.lax.broadcasted_iota(jnp.int32, logits.shape, 1)
    l1 = jnp.max(logits, axis=-1, keepdims=True)
    a1 = jnp.min(jnp.where(logits == l1, iota, e), axis=-1, keepdims=True)
    masked = jnp.where(iota == a1, -jnp.inf, logits)
    l2 = jnp.max(masked, axis=-1, keepdims=True)
    a2 = jnp.min(jnp.where(masked == l2, iota, e), axis=-1, keepdims=True)
    z = jnp.sum(jnp.exp(logits - l1), axis=-1, keepdims=True)
    p1 = 1.0 / z
    p2 = jnp.exp(l2 - l1) / z
    e2 = jnp.exp(p2 - p1)
    inv = 1.0 / (1.0 + e2)
    w1c_ref[...] = jnp.where(iota == a1, inv, 0.0)
    w2c_ref[...] = jnp.where(iota == a2, e2 * inv, 0.0)


def _k1b_body(w1c_ref, w2c_ref, b1a_ref, w2a_ref, b2a_ref,
              dst_ref, wt_ref, pos_ref, be_ref, ctot_ref,
              *, n_tok, n_exp, re_blk, n_blk):
    eye = (jax.lax.broadcasted_iota(_I32, (n_exp, n_exp), 0) ==
           jax.lax.broadcasted_iota(_I32, (n_exp, n_exp), 1)).astype(_F32)
    w1t = _dot_t(eye, w1c_ref[...])
    w2t = _dot_t(eye, w2c_ref[...])
    wt = w1t + w2t
    wt_ref[...] = wt

    m = (wt > 0.0).astype(_F32)
    incl = m
    s = 1
    while s < n_tok:
        shifted = jnp.concatenate(
            [jnp.zeros((n_exp, s), _F32), incl[:, :n_tok - s]], axis=1)
        incl = incl + shifted
        s *= 2
    cnt = incl[:, n_tok - 1:n_tok]
    pcnt = jnp.floor((cnt + (re_blk - 1)) * (1.0 / re_blk)) * re_blk
    mlt = (jax.lax.broadcasted_iota(_I32, (n_exp, n_exp), 1) <
           jax.lax.broadcasted_iota(_I32, (n_exp, n_exp), 0)).astype(_F32)
    offp = _dot(mlt, pcnt)
    dstf = offp + incl - 1.0
    dst_ref[...] = dstf.astype(_I32)

    pos_ref[0:1, :] = jnp.sum((w1t > 0.0).astype(_F32) * dstf, axis=0,
                              keepdims=True).astype(_I32)
    pos_ref[1:2, :] = jnp.sum((w2t > 0.0).astype(_F32) * dstf, axis=0,
                              keepdims=True).astype(_I32)

    biota = jax.lax.broadcasted_iota(_I32, (n_exp, n_blk), 1).astype(_F32) * re_blk
    be_ref[...] = (jnp.sum((biota >= offp).astype(_F32), axis=0,
                           keepdims=True) - 1.0).astype(_I32)

    ctot = jnp.zeros((1, ctot_ref.shape[1]), _F32)
    for i in range(n_exp):
        gb1 = _gelu(b1a_ref[i])
        ctot = ctot + _dot_t(gb1, w2a_ref[i]) + b2a_ref[i]
    ctot_ref[...] = ctot


def _sca_body(dst_hbm, gate_hbm, tok_out, gate_out,
              dstv, gatev, tokl, gatel, *, npad, n_entries, tokmask):
    own = npad // _NW
    wid = lax.axis_index("s") * _NC + lax.axis_index("c")
    lo = wid * own

    zi = jnp.zeros((_LANES,), _I32)
    zf = jnp.zeros((_LANES,), _F32)

    @pl.loop(0, own // _LANES)
    def _init(i):
        tokl[pl.ds(i * _LANES, _LANES)] = zi
        gatel[pl.ds(i * _LANES, _LANES)] = zf

    lane_iota = jax.lax.iota(_I32, _LANES)
    ch = dstv.shape[0]

    @pl.loop(0, n_entries // ch)
    def _chunk(c):
        base = c * ch
        pltpu.sync_copy(dst_hbm.at[pl.ds(base, ch)], dstv)
        pltpu.sync_copy(gate_hbm.at[pl.ds(base, ch)], gatev)

        @pl.loop(0, ch // _LANES)
        def _scan(i):
            dv = dstv[pl.ds(i * _LANES, _LANES)]
            gv = gatev[pl.ds(i * _LANES, _LANES)]
            lidx = dv - lo
            ok = (gv > 0.0) & (lidx >= 0) & (lidx < own)
            lidx = jnp.where(ok, lidx, 0)
            tid = jnp.bitwise_and(base + i * _LANES + lane_iota, tokmask)
            plsc.store_scatter(tokl, [lidx], tid, mask=ok)
            plsc.store_scatter(gatel, [lidx], gv, mask=ok)

    pltpu.sync_copy(tokl.at[pl.ds(0, own)], tok_out.at[pl.ds(lo, own)])
    pltpu.sync_copy(gatel.at[pl.ds(0, own)], gate_out.at[pl.ds(lo, own)])


def _scg_body(table_hbm, idx_hbm, out_hbm, idxv, rows, sem, *, n_rows, nmax):
    own = n_rows // _NW
    ch = idxv.shape[0]
    wid = lax.axis_index("s") * _NC + lax.axis_index("c")
    base = wid * own
    for c in range(own // ch):
        pltpu.sync_copy(idx_hbm.at[pl.ds(base + c * ch, ch)], idxv)
        if nmax is not None:
            @pl.loop(0, ch // _LANES)
            def _san(i):
                v = idxv[pl.ds(i * _LANES, _LANES)]
                idxv[pl.ds(i * _LANES, _LANES)] = jnp.minimum(
                    jnp.maximum(v, 0), nmax)
        pltpu.async_copy(table_hbm.at[idxv], rows, sem).wait()
        pltpu.sync_copy(rows, out_hbm.at[pl.ds(base + c * ch, ch)])


def _tce_body(be_ref, gbuf_ref, gate_ref, w1_ref, b1_ref, w2_ref, out_ref):
    b1 = b1_ref[0]
    z = _dot_t(gbuf_ref[...], w1_ref[0]) + b1
    t = _gelu(z) - _gelu(b1)
    out_ref[...] = _dot_t(gate_ref[...] * t, w2_ref[0])


def _k2_body(q_ref, k_ref, v_ref, x_ref, wo_ref, o_ref, *, scale):
    qb = q_ref[0]
    scores = _dot_t(qb, k_ref[0]) * scale
    m = jnp.max(scores, axis=-1, keepdims=True)
    p = jnp.exp(scores - m)
    att = p / jnp.sum(p, axis=-1, keepdims=True)
    o = _dot(att, v_ref[0])
    o_ref[0] = x_ref[0] + _dot_t(o, wo_ref[...])


def _k3_body(y1_ref, h_ref, g_ref, y2a_ref, y2b_ref, ctot_ref,
             ws1_ref, bs1_ref, ws2_ref, bs2_ref, wu_ref, bu_ref,
             wc_ref, bc_ref, out_ref):
    gb = g_ref[...]
    s = _gelu(_dot_t(gb, ws1_ref[...]) + bs1_ref[...])
    s = _dot_t(s, ws2_ref[...]) + bs2_ref[...]
    moe = y2a_ref[...] + y2b_ref[...] + ctot_ref[...] + 0.1 * s

    up = _dot_t(moe, wu_ref[...]) + bu_ref[...]
    core = _dot_t(_gelu(h_ref[...]), wc_ref[...]) + bc_ref[...]
    out_ref[...] = y1_ref[...] + up + core


def kernel(x, rms_w, Wqkv, Wo, Wd, bd, Wu, bu, Wr, br, expert_params,
           Ws1, bs1, Ws2, bs2, Wc, bc):
    B, T, D = x.shape
    N = B * T
    L = Wd.shape[0]
    E = Wr.shape[0]
    hdims = [int(w1.shape[0]) for (w1, _, _, _) in expert_params]
    hpad = max(hdims)
    cost = jnp.asarray([2 * L * hd for hd in hdims], _F32)
    br_eff = (br - _COST_LAMBDA * cost).reshape(1, E)

    RE = 128
    NPAD = 2 * N + E * RE
    NBLK = NPAD // RE

    W1all = jnp.stack([jnp.pad(w1, ((0, hpad - hd), (0, 0)))
                       for (w1, _, _, _), hd in zip(expert_params, hdims)])
    b1all = jnp.stack([jnp.pad(b1, (0, hpad - hd)).reshape(1, hpad)
                       for (_, b1, _, _), hd in zip(expert_params, hdims)])
    W2all = jnp.stack([jnp.pad(w2, ((0, 0), (0, hpad - hd)))
                       for (_, _, w2, _), hd in zip(expert_params, hdims)])
    b2all = jnp.stack([b2.reshape(1, L) for (_, _, _, b2) in expert_params])

    x2 = x.reshape(N, D)
    r2 = lambda a: a.reshape(1, -1)

    R1 = 512
    full = lambda arr: pl.BlockSpec(arr.shape, lambda i: (0,) * arr.ndim)
    row = lambda c: pl.BlockSpec((R1, c), lambda i: (i, 0))

    h, q, k, v, g, w1c, w2c = pl.pallas_call(
        _k1_body,
        grid=(N // R1,),
        in_specs=[row(D), full(r2(rms_w)), full(Wqkv), full(Wd),
                  full(r2(bd)), full(Wr), full(br_eff)],
        out_specs=[row(D), row(D), row(D), row(D), row(L), row(E), row(E)],
        out_shape=[
            jax.ShapeDtypeStruct((N, D), _F32),
            jax.ShapeDtypeStruct((N, D), _BF16),
            jax.ShapeDtypeStruct((N, D), _BF16),
            jax.ShapeDtypeStruct((N, D), _BF16),
            jax.ShapeDtypeStruct((N, L), _F32),
            jax.ShapeDtypeStruct((N, E), _F32),
            jax.ShapeDtypeStruct((N, E), _F32),
        ],
    )(x2, r2(rms_w), Wqkv, Wd, r2(bd), Wr, br_eff)

    dst8, wt8, pos2, be, ctot = pl.pallas_call(
        functools.partial(_k1b_body, n_tok=N, n_exp=E, re_blk=RE, n_blk=NBLK),
        grid=(1,),
        in_specs=[full(w1c), full(w2c), full(b1all), full(W2all), full(b2all)],
        out_specs=[
            pl.BlockSpec((E, N), lambda i: (0, 0)),
            pl.BlockSpec((E, N), lambda i: (0, 0)),
            pl.BlockSpec((2, N), lambda i: (0, 0)),
            pl.BlockSpec((1, NBLK), lambda i: (0, 0)),
            pl.BlockSpec((1, L), lambda i: (0, 0)),
        ],
        out_shape=[
            jax.ShapeDtypeStruct((E, N), _I32),
            jax.ShapeDtypeStruct((E, N), _F32),
            jax.ShapeDtypeStruct((2, N), _I32),
            jax.ShapeDtypeStruct((1, NBLK), _I32),
            jax.ShapeDtypeStruct((1, L), _F32),
        ],
    )(w1c, w2c, b1all, W2all, b2all)

    mesh = plsc.VectorSubcoreMesh(core_axis_name="c", subcore_axis_name="s")

    NENT = E * N
    SCCH = 4096
    sca = pl.kernel(
        functools.partial(_sca_body, npad=NPAD, n_entries=NENT,
                          tokmask=N - 1),
        out_type=[jax.ShapeDtypeStruct((NPAD,), _I32),
                  jax.ShapeDtypeStruct((NPAD,), _F32)],
        mesh=mesh,
        scratch_types=[pltpu.VMEM((SCCH,), _I32),
                       pltpu.VMEM((SCCH,), _F32),
                       pltpu.VMEM((_rup128(NPAD // _NW),), _I32),
                       pltpu.VMEM((_rup128(NPAD // _NW),), _F32)],
        compiler_params=pltpu.CompilerParams(needs_layout_passes=False),
    )
    tok, gateb = sca(dst8.reshape(NENT), wt8.reshape(NENT))

    RQ = 512
    q3 = q.reshape(B, T, D)
    k3 = k.reshape(B, T, D)
    v3 = v.reshape(B, T, D)
    qblk = pl.BlockSpec((1, RQ, D), lambda b, i: (b, i, 0))
    kvblk = pl.BlockSpec((1, T, D), lambda b, i: (b, 0, 0))
    y1 = pl.pallas_call(
        functools.partial(_k2_body, scale=D ** -0.5),
        grid=(B, T // RQ),
        in_specs=[qblk, kvblk, kvblk, qblk,
                  pl.BlockSpec(Wo.shape, lambda b, i: (0, 0))],
        out_specs=qblk,
        out_shape=jax.ShapeDtypeStruct((B, T, D), _F32),
    )(q3, k3, v3, x, Wo)

    scb = pl.kernel(
        functools.partial(_scg_body, n_rows=NPAD, nmax=None),
        out_type=jax.ShapeDtypeStruct((NPAD, L), _F32),
        mesh=mesh,
        scratch_types=[pltpu.VMEM((NPAD // _NW // 2,), _I32),
                       pltpu.VMEM((NPAD // _NW // 2, L), _F32),
                       pltpu.SemaphoreType.DMA],
    )
    gbuf = scb(g, tok)

    ybuf = pl.pallas_call(
        _tce_body,
        grid_spec=pltpu.PrefetchScalarGridSpec(
            num_scalar_prefetch=1,
            grid=(NBLK,),
            in_specs=[
                pl.BlockSpec((RE, L), lambda b, be_s: (b, 0)),
                pl.BlockSpec((RE, 1), lambda b, be_s: (b, 0)),
                pl.BlockSpec((1, hpad, L), lambda b, be_s: (be_s[b], 0, 0)),
                pl.BlockSpec((1, 1, hpad), lambda b, be_s: (be_s[b], 0, 0)),
                pl.BlockSpec((1, L, hpad), lambda b, be_s: (be_s[b], 0, 0)),
            ],
            out_specs=pl.BlockSpec((RE, L), lambda b, be_s: (b, 0)),
        ),
        out_shape=jax.ShapeDtypeStruct((NPAD, L), _F32),
    )(be.reshape(NBLK), gbuf, gateb.reshape(NPAD, 1), W1all, b1all, W2all)

    scc = pl.kernel(
        functools.partial(_scg_body, n_rows=2 * N, nmax=None),
        out_type=jax.ShapeDtypeStruct((2 * N, L), _F32),
        mesh=mesh,
        scratch_types=[pltpu.VMEM((2 * N // _NW // 2,), _I32),
                       pltpu.VMEM((2 * N // _NW // 2, L), _F32),
                       pltpu.SemaphoreType.DMA],
    )
    y2 = scc(ybuf, pos2.reshape(2 * N))

    R3 = 512
    row3 = lambda c: pl.BlockSpec((R3, c), lambda i: (i, 0))
    out = pl.pallas_call(
        _k3_body,
        grid=(N // R3,),
        in_specs=[row3(D), row3(D), row3(L), row3(L), row3(L),
                  full(ctot),
                  full(Ws1), full(r2(bs1)), full(Ws2), full(r2(bs2)),
                  full(Wu), full(r2(bu)), full(Wc), full(r2(bc))],
        out_specs=row3(D),
        out_shape=jax.ShapeDtypeStruct((N, D), _F32),
    )(y1.reshape(N, D), h, g, y2[:N], y2[N:], ctot,
      Ws1, r2(bs1), Ws2, r2(bs2), Wu, r2(bu), Wc, r2(bc))

    return out.reshape(B, T, D)


# speedup vs baseline: 1.7713x; 1.0102x over previous
import functools

import jax
import jax.numpy as jnp
from jax import lax
from jax.experimental import pallas as pl
from jax.experimental.pallas import tpu as pltpu
from jax.experimental.pallas import tpu_sc as plsc

_F32 = jnp.float32
_BF16 = jnp.bfloat16
_I32 = jnp.int32
_COST_LAMBDA = 0.0005

_NC, _NS, _LANES = 2, 16, 16
_NW = _NC * _NS


def _rup128(n):
    return (n + 127) // 128 * 128


def _gelu(v):
    return 0.5 * v * (1.0 + jax.lax.erf(v * (2.0 ** -0.5)))


def _dot_t(a, b):
    return jax.lax.dot_general(a, b, (((1,), (1,)), ((), ())),
                               preferred_element_type=_F32)


def _dot(a, b):
    return jax.lax.dot_general(a, b, (((1,), (0,)), ((), ())),
                               preferred_element_type=_F32)


def _k1_body(x_ref, rmsw_ref, wqkv_ref, wd_ref, bd_ref, wr_ref, breff_ref,
             h_ref, q_ref, k_ref, v_ref, g_ref, w1c_ref, w2c_ref):
    xb = x_ref[...]
    d = xb.shape[-1]
    norm = jnp.sqrt(jnp.sum(xb * xb, axis=-1, keepdims=True)) * (d ** -0.5)
    hb = rmsw_ref[...] * xb / (norm + 1e-8)
    h_ref[...] = hb

    qkv = _dot_t(hb, wqkv_ref[...])
    q_ref[...] = qkv[:, :d].astype(_BF16)
    k_ref[...] = qkv[:, d:2 * d].astype(_BF16)
    v_ref[...] = qkv[:, 2 * d:].astype(_BF16)

    hd = _dot_t(hb, wd_ref[...]) + bd_ref[...]
    g_ref[...] = _gelu(hd)

    logits = _dot_t(hb, wr_ref[...]) + breff_ref[...]
    e = logits.shape[-1]
    iota = jax.lax.broadcasted_iota(jnp.int32, logits.shape, 1)
    l1 = jnp.max(logits, axis=-1, keepdims=True)
    a1 = jnp.min(jnp.where(logits == l1, iota, e), axis=-1, keepdims=True)
    masked = jnp.where(iota == a1, -jnp.inf, logits)
    l2 = jnp.max(masked, axis=-1, keepdims=True)
    a2 = jnp.min(jnp.where(masked == l2, iota, e), axis=-1, keepdims=True)
    z = jnp.sum(jnp.exp(logits - l1), axis=-1, keepdims=True)
    p1 = 1.0 / z
    p2 = jnp.exp(l2 - l1) / z
    e2 = jnp.exp(p2 - p1)
    inv = 1.0 / (1.0 + e2)
    w1c_ref[...] = jnp.where(iota == a1, inv, 0.0)
    w2c_ref[...] = jnp.where(iota == a2, e2 * inv, 0.0)


def _k1b_body(w1c_ref, w2c_ref, b1a_ref, w2a_ref, b2a_ref,
              dst_ref, wt_ref, pos_ref, be_ref, ctot_ref,
              *, n_tok, n_exp, re_blk, n_blk):
    eye = (jax.lax.broadcasted_iota(_I32, (n_exp, n_exp), 0) ==
           jax.lax.broadcasted_iota(_I32, (n_exp, n_exp), 1)).astype(_F32)
    w1t = _dot_t(eye, w1c_ref[...])
    w2t = _dot_t(eye, w2c_ref[...])
    wt = w1t + w2t
    wt_ref[...] = wt

    m = (wt > 0.0).astype(_F32)
    incl = m
    s = 1
    while s < n_tok:
        shifted = jnp.concatenate(
            [jnp.zeros((n_exp, s), _F32), incl[:, :n_tok - s]], axis=1)
        incl = incl + shifted
        s *= 2
    cnt = incl[:, n_tok - 1:n_tok]
    pcnt = jnp.floor((cnt + (re_blk - 1)) * (1.0 / re_blk)) * re_blk
    mlt = (jax.lax.broadcasted_iota(_I32, (n_exp, n_exp), 1) <
           jax.lax.broadcasted_iota(_I32, (n_exp, n_exp), 0)).astype(_F32)
    offp = _dot(mlt, pcnt)
    dstf = offp + incl - 1.0
    dst_ref[...] = dstf.astype(_I32)

    pos_ref[0:1, :] = jnp.sum((w1t > 0.0).astype(_F32) * dstf, axis=0,
                              keepdims=True).astype(_I32)
    pos_ref[1:2, :] = jnp.sum((w2t > 0.0).astype(_F32) * dstf, axis=0,
                              keepdims=True).astype(_I32)

    biota = jax.lax.broadcasted_iota(_I32, (n_exp, n_blk), 1).astype(_F32) * re_blk
    be_ref[...] = (jnp.sum((biota >= offp).astype(_F32), axis=0,
                           keepdims=True) - 1.0).astype(_I32)

    ctot = jnp.zeros((1, ctot_ref.shape[1]), _F32)
    for i in range(n_exp):
        gb1 = _gelu(b1a_ref[i])
        ctot = ctot + _dot_t(gb1, w2a_ref[i]) + b2a_ref[i]
    ctot_ref[...] = ctot


def _sca_body(dst_hbm, gate_hbm, tok_out, gate_out,
              dstv, gatev, tokl, gatel, *, npad, n_entries, tokmask):
    own = npad // _NW
    wid = lax.axis_index("s") * _NC + lax.axis_index("c")
    lo = wid * own

    zi = jnp.zeros((_LANES,), _I32)
    zf = jnp.zeros((_LANES,), _F32)

    @pl.loop(0, own // _LANES)
    def _init(i):
        tokl[pl.ds(i * _LANES, _LANES)] = zi
        gatel[pl.ds(i * _LANES, _LANES)] = zf

    lane_iota = jax.lax.iota(_I32, _LANES)
    ch = dstv.shape[0]

    @pl.loop(0, n_entries // ch)
    def _chunk(c):
        base = c * ch
        pltpu.sync_copy(dst_hbm.at[pl.ds(base, ch)], dstv)
        pltpu.sync_copy(gate_hbm.at[pl.ds(base, ch)], gatev)

        @pl.loop(0, ch // _LANES)
        def _scan(i):
            dv = dstv[pl.ds(i * _LANES, _LANES)]
            gv = gatev[pl.ds(i * _LANES, _LANES)]
            lidx = dv - lo
            ok = (gv > 0.0) & (lidx >= 0) & (lidx < own)
            lidx = jnp.where(ok, lidx, 0)
            tid = jnp.bitwise_and(base + i * _LANES + lane_iota, tokmask)
            plsc.store_scatter(tokl, [lidx], tid, mask=ok)
            plsc.store_scatter(gatel, [lidx], gv, mask=ok)

    pltpu.sync_copy(tokl.at[pl.ds(0, own)], tok_out.at[pl.ds(lo, own)])
    pltpu.sync_copy(gatel.at[pl.ds(0, own)], gate_out.at[pl.ds(lo, own)])


def _scg_body(table_hbm, idx_hbm, out_hbm, idxv0, idxv1, rows0, rows1,
              sem0, sem1, *, n_rows):
    own = n_rows // _NW
    ch = rows0.shape[0]
    wid = lax.axis_index("s") * _NC + lax.axis_index("c")
    base = wid * own
    bufs = ((idxv0, rows0, sem0), (idxv1, rows1, sem1))
    n = own // ch
    pltpu.sync_copy(idx_hbm.at[pl.ds(base, ch)], idxv0)
    cps = [pltpu.async_copy(table_hbm.at[idxv0], rows0, sem0), None]
    for c in range(n):
        cur = bufs[c % 2]
        nxt = bufs[(c + 1) % 2]
        if c + 1 < n:
            pltpu.sync_copy(idx_hbm.at[pl.ds(base + (c + 1) * ch, ch)], nxt[0])
        cps[c % 2].wait()
        if c + 1 < n:
            cps[(c + 1) % 2] = pltpu.async_copy(
                table_hbm.at[nxt[0]], nxt[1], nxt[2])
        pltpu.sync_copy(cur[1], out_hbm.at[pl.ds(base + c * ch, ch)])


def _tce_body(be_ref, gbuf_ref, gate_ref, w1_ref, b1_ref, w2_ref, out_ref):
    b1 = b1_ref[0]
    z = _dot_t(gbuf_ref[...], w1_ref[0]) + b1
    t = _gelu(z) - _gelu(b1)
    out_ref[...] = _dot_t(gate_ref[...] * t, w2_ref[0])


def _k2_body(q_ref, k_ref, v_ref, x_ref, wo_ref, o_ref, *, scale):
    qb = q_ref[0]
    scores = _dot_t(qb, k_ref[0]) * scale
    m = jnp.max(scores, axis=-1, keepdims=True)
    p = jnp.exp(scores - m)
    att = p / jnp.sum(p, axis=-1, keepdims=True)
    o = _dot(att, v_ref[0])
    o_ref[0] = x_ref[0] + _dot_t(o, wo_ref[...])


def _k3_body(y1_ref, h_ref, g_ref, y2a_ref, y2b_ref, ctot_ref,
             ws1_ref, bs1_ref, ws2_ref, bs2_ref, wu_ref, bu_ref,
             wc_ref, bc_ref, out_ref):
    gb = g_ref[...]
    s = _gelu(_dot_t(gb, ws1_ref[...]) + bs1_ref[...])
    s = _dot_t(s, ws2_ref[...]) + bs2_ref[...]
    moe = y2a_ref[...] + y2b_ref[...] + ctot_ref[...] + 0.1 * s

    up = _dot_t(moe, wu_ref[...]) + bu_ref[...]
    core = _dot_t(_gelu(h_ref[...]), wc_ref[...]) + bc_ref[...]
    out_ref[...] = y1_ref[...] + up + core


def kernel(x, rms_w, Wqkv, Wo, Wd, bd, Wu, bu, Wr, br, expert_params,
           Ws1, bs1, Ws2, bs2, Wc, bc):
    B, T, D = x.shape
    N = B * T
    L = Wd.shape[0]
    E = Wr.shape[0]
    hdims = [int(w1.shape[0]) for (w1, _, _, _) in expert_params]
    hpad = max(hdims)
    cost = jnp.asarray([2 * L * hd for hd in hdims], _F32)
    br_eff = (br - _COST_LAMBDA * cost).reshape(1, E)

    RE = 128
    NPAD = 2 * N + E * RE
    NBLK = NPAD // RE

    W1all = jnp.stack([jnp.pad(w1, ((0, hpad - hd), (0, 0)))
                       for (w1, _, _, _), hd in zip(expert_params, hdims)])
    b1all = jnp.stack([jnp.pad(b1, (0, hpad - hd)).reshape(1, hpad)
                       for (_, b1, _, _), hd in zip(expert_params, hdims)])
    W2all = jnp.stack([jnp.pad(w2, ((0, 0), (0, hpad - hd)))
                       for (_, _, w2, _), hd in zip(expert_params, hdims)])
    b2all = jnp.stack([b2.reshape(1, L) for (_, _, _, b2) in expert_params])

    x2 = x.reshape(N, D)
    r2 = lambda a: a.reshape(1, -1)

    R1 = 512
    full = lambda arr: pl.BlockSpec(arr.shape, lambda i: (0,) * arr.ndim)
    row = lambda c: pl.BlockSpec((R1, c), lambda i: (i, 0))

    h, q, k, v, g, w1c, w2c = pl.pallas_call(
        _k1_body,
        grid=(N // R1,),
        in_specs=[row(D), full(r2(rms_w)), full(Wqkv), full(Wd),
                  full(r2(bd)), full(Wr), full(br_eff)],
        out_specs=[row(D), row(D), row(D), row(D), row(L), row(E), row(E)],
        out_shape=[
            jax.ShapeDtypeStruct((N, D), _F32),
            jax.ShapeDtypeStruct((N, D), _BF16),
            jax.ShapeDtypeStruct((N, D), _BF16),
            jax.ShapeDtypeStruct((N, D), _BF16),
            jax.ShapeDtypeStruct((N, L), _F32),
            jax.ShapeDtypeStruct((N, E), _F32),
            jax.ShapeDtypeStruct((N, E), _F32),
        ],
    )(x2, r2(rms_w), Wqkv, Wd, r2(bd), Wr, br_eff)

    dst8, wt8, pos2, be, ctot = pl.pallas_call(
        functools.partial(_k1b_body, n_tok=N, n_exp=E, re_blk=RE, n_blk=NBLK),
        grid=(1,),
        in_specs=[full(w1c), full(w2c), full(b1all), full(W2all), full(b2all)],
        out_specs=[
            pl.BlockSpec((E, N), lambda i: (0, 0)),
            pl.BlockSpec((E, N), lambda i: (0, 0)),
            pl.BlockSpec((2, N), lambda i: (0, 0)),
            pl.BlockSpec((1, NBLK), lambda i: (0, 0)),
            pl.BlockSpec((1, L), lambda i: (0, 0)),
        ],
        out_shape=[
            jax.ShapeDtypeStruct((E, N), _I32),
            jax.ShapeDtypeStruct((E, N), _F32),
            jax.ShapeDtypeStruct((2, N), _I32),
            jax.ShapeDtypeStruct((1, NBLK), _I32),
            jax.ShapeDtypeStruct((1, L), _F32),
        ],
    )(w1c, w2c, b1all, W2all, b2all)

    mesh = plsc.VectorSubcoreMesh(core_axis_name="c", subcore_axis_name="s")

    NENT = E * N
    SCCH = 4096
    sca = pl.kernel(
        functools.partial(_sca_body, npad=NPAD, n_entries=NENT,
                          tokmask=N - 1),
        out_type=[jax.ShapeDtypeStruct((NPAD,), _I32),
                  jax.ShapeDtypeStruct((NPAD,), _F32)],
        mesh=mesh,
        scratch_types=[pltpu.VMEM((SCCH,), _I32),
                       pltpu.VMEM((SCCH,), _F32),
                       pltpu.VMEM((_rup128(NPAD // _NW),), _I32),
                       pltpu.VMEM((_rup128(NPAD // _NW),), _F32)],
        compiler_params=pltpu.CompilerParams(needs_layout_passes=False),
    )
    tok, gateb = sca(dst8.reshape(NENT), wt8.reshape(NENT))

    RQ = 512
    q3 = q.reshape(B, T, D)
    k3 = k.reshape(B, T, D)
    v3 = v.reshape(B, T, D)
    qblk = pl.BlockSpec((1, RQ, D), lambda b, i: (b, i, 0))
    kvblk = pl.BlockSpec((1, T, D), lambda b, i: (b, 0, 0))
    y1 = pl.pallas_call(
        functools.partial(_k2_body, scale=D ** -0.5),
        grid=(B, T // RQ),
        in_specs=[qblk, kvblk, kvblk, qblk,
                  pl.BlockSpec(Wo.shape, lambda b, i: (0, 0))],
        out_specs=qblk,
        out_shape=jax.ShapeDtypeStruct((B, T, D), _F32),
    )(q3, k3, v3, x, Wo)

    scb = pl.kernel(
        functools.partial(_scg_body, n_rows=NPAD),
        out_type=jax.ShapeDtypeStruct((NPAD, L), _F32),
        mesh=mesh,
        scratch_types=[pltpu.VMEM((NPAD // _NW // 4,), _I32),
                       pltpu.VMEM((NPAD // _NW // 4,), _I32),
                       pltpu.VMEM((NPAD // _NW // 4, L), _F32),
                       pltpu.VMEM((NPAD // _NW // 4, L), _F32),
                       pltpu.SemaphoreType.DMA,
                       pltpu.SemaphoreType.DMA],
    )
    gbuf = scb(g, tok)

    ybuf = pl.pallas_call(
        _tce_body,
        grid_spec=pltpu.PrefetchScalarGridSpec(
            num_scalar_prefetch=1,
            grid=(NBLK,),
            in_specs=[
                pl.BlockSpec((RE, L), lambda b, be_s: (b, 0)),
                pl.BlockSpec((RE, 1), lambda b, be_s: (b, 0)),
                pl.BlockSpec((1, hpad, L), lambda b, be_s: (be_s[b], 0, 0)),
                pl.BlockSpec((1, 1, hpad), lambda b, be_s: (be_s[b], 0, 0)),
                pl.BlockSpec((1, L, hpad), lambda b, be_s: (be_s[b], 0, 0)),
            ],
            out_specs=pl.BlockSpec((RE, L), lambda b, be_s: (b, 0)),
        ),
        out_shape=jax.ShapeDtypeStruct((NPAD, L), _F32),
    )(be.reshape(NBLK), gbuf, gateb.reshape(NPAD, 1), W1all, b1all, W2all)

    scc = pl.kernel(
        functools.partial(_scg_body, n_rows=2 * N),
        out_type=jax.ShapeDtypeStruct((2 * N, L), _F32),
        mesh=mesh,
        scratch_types=[pltpu.VMEM((2 * N // _NW // 4,), _I32),
                       pltpu.VMEM((2 * N // _NW // 4,), _I32),
                       pltpu.VMEM((2 * N // _NW // 4, L), _F32),
                       pltpu.VMEM((2 * N // _NW // 4, L), _F32),
                       pltpu.SemaphoreType.DMA,
                       pltpu.SemaphoreType.DMA],
    )
    y2 = scc(ybuf, pos2.reshape(2 * N))

    R3 = 512
    row3 = lambda c: pl.BlockSpec((R3, c), lambda i: (i, 0))
    out = pl.pallas_call(
        _k3_body,
        grid=(N // R3,),
        in_specs=[row3(D), row3(D), row3(L), row3(L), row3(L),
                  full(ctot),
                  full(Ws1), full(r2(bs1)), full(Ws2), full(r2(bs2)),
                  full(Wu), full(r2(bu)), full(Wc), full(r2(bc))],
        out_specs=row3(D),
        out_shape=jax.ShapeDtypeStruct((N, D), _F32),
    )(y1.reshape(N, D), h, g, y2[:N], y2[N:], ctot,
      Ws1, r2(bs1), Ws2, r2(bs2), Wu, r2(bu), Wc, r2(bc))

    return out.reshape(B, T, D)
